# baseline (device time: 38748 ns/iter reference)
import jax
import jax.numpy as jnp
from jax import lax
from jax.experimental import pallas as pl
from jax.experimental.pallas import tpu as pltpu

N_DEV = 16
SQ = 256
D = 1024
DH = 128
NH_LOCAL = 8
CH = SQ // N_DEV
SCALE = 0.08838834764831843


def kernel(x, Wq, Wo, Wk, Wv):
    def body(
        x_ref,
        wq_ref,
        wo_ref,
        wk_ref,
        wv_ref,
        out_ref,
        x_vm,
        wq_vm,
        wo_vm,
        wk_vm,
        wv_vm,
        out_vm,
        part16_ref,
        rs_buf,
        ag_buf,
        load_sems,
        rs_send_sems,
        rs_recv_sems,
        ag_send_sems,
        ag_recv_sems,
    ):
        p = lax.axis_index("i")
        bf16 = jnp.bfloat16

        loads = []
        for idx, (hbm, vm) in enumerate(
            [(x_ref, x_vm), (wk_ref, wk_vm), (wv_ref, wv_vm),
             (wq_ref, wq_vm), (wo_ref, wo_vm)]
        ):
            cp = pltpu.make_async_copy(hbm, vm, load_sems.at[idx])
            cp.start()
            loads.append(cp)
        load_x, load_wk, load_wv, load_wq, load_wo = loads

        load_x.wait()
        xm = x_vm[0].astype(bf16)
        load_wk.wait()
        k16 = jnp.dot(
            xm, wk_vm[...].astype(bf16), preferred_element_type=jnp.float32
        ).astype(bf16)
        load_wv.wait()
        v16 = jnp.dot(
            xm, wv_vm[...].astype(bf16), preferred_element_type=jnp.float32
        ).astype(bf16)
        load_wq.wait()
        q16 = jnp.dot(
            xm, wq_vm[...].astype(bf16), preferred_element_type=jnp.float32
        ).astype(bf16)

        part = jnp.zeros((SQ, D), jnp.float32)
        wo16 = None
        for h in range(NH_LOCAL):
            sl = slice(h * DH, (h + 1) * DH)
            s = (
                lax.dot_general(
                    q16[:, sl],
                    k16[:, sl],
                    (((1,), (1,)), ((), ())),
                    preferred_element_type=jnp.float32,
                )
                * SCALE
            )
            m = jnp.max(s, axis=1, keepdims=True)
            pr = jnp.exp(s - m)
            l = jnp.sum(pr, axis=1, keepdims=True)
            o = jnp.dot(
                pr.astype(bf16), v16[:, sl], preferred_element_type=jnp.float32
            ) / l
            if wo16 is None:
                load_wo.wait()
                wo16 = wo_vm[...].astype(bf16)
            part = part + jnp.dot(
                o.astype(bf16), wo16[sl, :], preferred_element_type=jnp.float32
            )
        part16_ref[...] = part.astype(bf16)
        rs_buf[p, :, :] = part16_ref[pl.ds(p * CH, CH), :]

        barrier = pltpu.get_barrier_semaphore()
        for j in range(N_DEV - 1):
            pl.semaphore_signal(
                barrier,
                inc=1,
                device_id=(lax.rem(p + 1 + j, N_DEV),),
                device_id_type=pl.DeviceIdType.MESH,
            )
        pl.semaphore_wait(barrier, N_DEV - 1)

        rs_rdmas = []
        for j in range(N_DEV - 1):
            tgt = lax.rem(p + 1 + j, N_DEV)
            rdma = pltpu.make_async_remote_copy(
                src_ref=part16_ref.at[pl.ds(tgt * CH, CH), :],
                dst_ref=rs_buf.at[p],
                send_sem=rs_send_sems.at[j],
                recv_sem=rs_recv_sems.at[j],
                device_id=(tgt,),
                device_id_type=pl.DeviceIdType.MESH,
            )
            rdma.start()
            rs_rdmas.append(rdma)
        for rdma in rs_rdmas:
            rdma.wait_recv()

        red = rs_buf[0].astype(jnp.float32)
        for s_ in range(1, N_DEV):
            red = red + rs_buf[s_].astype(jnp.float32)
        myrows = pl.ds(p * CH, CH)
        ag_buf[myrows, :] = red.astype(bf16)

        ag_rdmas = []
        for j in range(N_DEV - 1):
            tgt = lax.rem(p + 1 + j, N_DEV)
            rdma = pltpu.make_async_remote_copy(
                src_ref=ag_buf.at[myrows, :],
                dst_ref=ag_buf.at[myrows, :],
                send_sem=ag_send_sems.at[j],
                recv_sem=ag_recv_sems.at[j],
                device_id=(tgt,),
                device_id_type=pl.DeviceIdType.MESH,
            )
            rdma.start()
            ag_rdmas.append(rdma)
        for rdma in ag_rdmas:
            rdma.wait_recv()

        out_vm[0] = ag_buf[...].astype(jnp.float32)
        out_vm[0, myrows, :] = red
        store = pltpu.make_async_copy(out_vm, out_ref, load_sems.at[5])
        store.start()
        store.wait()

        for rdma in rs_rdmas:
            rdma.wait_send()
        for rdma in ag_rdmas:
            rdma.wait_send()

    return pl.pallas_call(
        body,
        out_shape=jax.ShapeDtypeStruct((1, SQ, D), jnp.float32),
        in_specs=[pl.BlockSpec(memory_space=pltpu.MemorySpace.HBM)] * 5,
        out_specs=pl.BlockSpec(memory_space=pltpu.MemorySpace.HBM),
        scratch_shapes=[
            pltpu.VMEM((1, SQ, D), jnp.float32),
            pltpu.VMEM((D, D), jnp.float32),
            pltpu.VMEM((D, D), jnp.float32),
            pltpu.VMEM((D, D), jnp.float32),
            pltpu.VMEM((D, D), jnp.float32),
            pltpu.VMEM((1, SQ, D), jnp.float32),
            pltpu.VMEM((SQ, D), jnp.bfloat16),
            pltpu.VMEM((N_DEV, CH, D), jnp.bfloat16),
            pltpu.VMEM((SQ, D), jnp.bfloat16),
            pltpu.SemaphoreType.DMA((6,)),
            pltpu.SemaphoreType.DMA((N_DEV - 1,)),
            pltpu.SemaphoreType.DMA((N_DEV - 1,)),
            pltpu.SemaphoreType.DMA((N_DEV - 1,)),
            pltpu.SemaphoreType.DMA((N_DEV - 1,)),
        ],
        compiler_params=pltpu.CompilerParams(collective_id=0),
    )(x, Wq, Wo, Wk, Wv)


# device time: 36657 ns/iter; 1.0570x vs baseline; 1.0570x over previous
import jax
import jax.numpy as jnp
from jax import lax
from jax.experimental import pallas as pl
from jax.experimental.pallas import tpu as pltpu

N_DEV = 16
SQ = 256
D = 1024
DH = 128
NH_LOCAL = 8
CH = SQ // N_DEV
SCALE = 0.08838834764831843


def kernel(x, Wq, Wo, Wk, Wv):
    def body(
        x_ref,
        wq_ref,
        wo_ref,
        wk_ref,
        wv_ref,
        out_ref,
        out_vm,
        part16_ref,
        rs_buf,
        ag_buf,
        store_sem,
        rs_send_sems,
        rs_recv_sems,
        ag_send_sems,
        ag_recv_sems,
    ):
        p = lax.axis_index("i")
        bf16 = jnp.bfloat16

        xm = x_ref[0].astype(bf16)
        q = jnp.dot(xm, wq_ref[...].astype(bf16), preferred_element_type=jnp.float32)
        k = jnp.dot(xm, wk_ref[...].astype(bf16), preferred_element_type=jnp.float32)
        v = jnp.dot(xm, wv_ref[...].astype(bf16), preferred_element_type=jnp.float32)
        q16 = q.astype(bf16)
        k16 = k.astype(bf16)
        v16 = v.astype(bf16)
        wo16 = wo_ref[...].astype(bf16)

        part = jnp.zeros((SQ, D), jnp.float32)
        for h in range(NH_LOCAL):
            sl = slice(h * DH, (h + 1) * DH)
            s = (
                lax.dot_general(
                    q16[:, sl],
                    k16[:, sl],
                    (((1,), (1,)), ((), ())),
                    preferred_element_type=jnp.float32,
                )
                * SCALE
            )
            m = jnp.max(s, axis=1, keepdims=True)
            pr = jnp.exp(s - m)
            l = jnp.sum(pr, axis=1, keepdims=True)
            o = jnp.dot(
                pr.astype(bf16), v16[:, sl], preferred_element_type=jnp.float32
            ) / l
            part = part + jnp.dot(
                o.astype(bf16), wo16[sl, :], preferred_element_type=jnp.float32
            )
        part16_ref[...] = part.astype(bf16)
        rs_buf[p, :, :] = part16_ref[pl.ds(p * CH, CH), :]

        barrier = pltpu.get_barrier_semaphore()
        for j in range(N_DEV - 1):
            pl.semaphore_signal(
                barrier,
                inc=1,
                device_id=(lax.rem(p + 1 + j, N_DEV),),
                device_id_type=pl.DeviceIdType.MESH,
            )
        pl.semaphore_wait(barrier, N_DEV - 1)

        rs_rdmas = []
        for j in range(N_DEV - 1):
            tgt = lax.rem(p + 1 + j, N_DEV)
            rdma = pltpu.make_async_remote_copy(
                src_ref=part16_ref.at[pl.ds(tgt * CH, CH), :],
                dst_ref=rs_buf.at[p],
                send_sem=rs_send_sems.at[j],
                recv_sem=rs_recv_sems.at[j],
                device_id=(tgt,),
                device_id_type=pl.DeviceIdType.MESH,
            )
            rdma.start()
            rs_rdmas.append(rdma)
        for rdma in rs_rdmas:
            rdma.wait_recv()

        red = rs_buf[0].astype(jnp.float32)
        for s_ in range(1, N_DEV):
            red = red + rs_buf[s_].astype(jnp.float32)
        myrows = pl.ds(p * CH, CH)
        ag_buf[myrows, :] = red.astype(bf16)

        ag_rdmas = []
        for j in range(N_DEV - 1):
            tgt = lax.rem(p + 1 + j, N_DEV)
            rdma = pltpu.make_async_remote_copy(
                src_ref=ag_buf.at[myrows, :],
                dst_ref=ag_buf.at[myrows, :],
                send_sem=ag_send_sems.at[j],
                recv_sem=ag_recv_sems.at[j],
                device_id=(tgt,),
                device_id_type=pl.DeviceIdType.MESH,
            )
            rdma.start()
            ag_rdmas.append(rdma)
        for rdma in ag_rdmas:
            rdma.wait_recv()

        out_vm[0] = ag_buf[...].astype(jnp.float32)
        out_vm[0, myrows, :] = red
        store = pltpu.make_async_copy(out_vm, out_ref, store_sem)
        store.start()
        store.wait()

        for rdma in rs_rdmas:
            rdma.wait_send()
        for rdma in ag_rdmas:
            rdma.wait_send()

    return pl.pallas_call(
        body,
        out_shape=jax.ShapeDtypeStruct((1, SQ, D), jnp.float32),
        in_specs=[pl.BlockSpec(memory_space=pltpu.VMEM)] * 5,
        out_specs=pl.BlockSpec(memory_space=pltpu.MemorySpace.HBM),
        scratch_shapes=[
            pltpu.VMEM((1, SQ, D), jnp.float32),
            pltpu.VMEM((SQ, D), jnp.bfloat16),
            pltpu.VMEM((N_DEV, CH, D), jnp.bfloat16),
            pltpu.VMEM((SQ, D), jnp.bfloat16),
            pltpu.SemaphoreType.DMA,
            pltpu.SemaphoreType.DMA((N_DEV - 1,)),
            pltpu.SemaphoreType.DMA((N_DEV - 1,)),
            pltpu.SemaphoreType.DMA((N_DEV - 1,)),
            pltpu.SemaphoreType.DMA((N_DEV - 1,)),
        ],
        compiler_params=pltpu.CompilerParams(collective_id=0),
    )(x, Wq, Wo, Wk, Wv)


# device time: 34613 ns/iter; 1.1195x vs baseline; 1.0591x over previous
import jax
import jax.numpy as jnp
from jax import lax
from jax.experimental import pallas as pl
from jax.experimental.pallas import tpu as pltpu

N_DEV = 16
SQ = 256
D = 1024
DH = 128
NH_LOCAL = 8
CH = SQ // N_DEV
SCALE = 0.08838834764831843


def kernel(x, Wq, Wo, Wk, Wv):
    def body(
        x_ref,
        wq_ref,
        wo_ref,
        wk_ref,
        wv_ref,
        out_ref,
        out_vm,
        part16_ref,
        rs_buf,
        ag_buf,
        store_sem,
        rs_send_sems,
        rs_recv_sems,
        ag_send_sems,
        ag_recv_sems,
    ):
        p = lax.axis_index("i")
        bf16 = jnp.bfloat16

        barrier = pltpu.get_barrier_semaphore()
        for j in range(N_DEV - 1):
            pl.semaphore_signal(
                barrier,
                inc=1,
                device_id=(lax.rem(p + 1 + j, N_DEV),),
                device_id_type=pl.DeviceIdType.MESH,
            )

        xm = x_ref[0].astype(bf16)
        q = jnp.dot(xm, wq_ref[...].astype(bf16), preferred_element_type=jnp.float32)
        k = jnp.dot(xm, wk_ref[...].astype(bf16), preferred_element_type=jnp.float32)
        v = jnp.dot(xm, wv_ref[...].astype(bf16), preferred_element_type=jnp.float32)
        q16 = q.astype(bf16)
        k16 = k.astype(bf16)
        v16 = v.astype(bf16)
        wo16 = wo_ref[...].astype(bf16)

        part = jnp.zeros((SQ, D), jnp.float32)
        for h in range(NH_LOCAL):
            sl = slice(h * DH, (h + 1) * DH)
            s = (
                lax.dot_general(
                    q16[:, sl],
                    k16[:, sl],
                    (((1,), (1,)), ((), ())),
                    preferred_element_type=jnp.float32,
                )
                * SCALE
            )
            m = jnp.max(s, axis=1, keepdims=True)
            pr = jnp.exp(s - m)
            l = jnp.sum(pr, axis=1, keepdims=True)
            o = jnp.dot(
                pr.astype(bf16), v16[:, sl], preferred_element_type=jnp.float32
            ) / l
            part = part + jnp.dot(
                o.astype(bf16), wo16[sl, :], preferred_element_type=jnp.float32
            )
        part16_ref[...] = part.astype(bf16)
        rs_buf[p, :, :] = part16_ref[pl.ds(p * CH, CH), :]

        pl.semaphore_wait(barrier, N_DEV - 1)

        HC = D // 2
        myrows = pl.ds(p * CH, CH)
        rs_rdmas = {0: [], 1: []}
        for half in (0, 1):
            cols = pl.ds(half * HC, HC)
            for j in range(N_DEV - 1):
                tgt = lax.rem(p + 1 + j, N_DEV)
                rdma = pltpu.make_async_remote_copy(
                    src_ref=part16_ref.at[pl.ds(tgt * CH, CH), cols],
                    dst_ref=rs_buf.at[p, :, cols],
                    send_sem=rs_send_sems.at[half, j],
                    recv_sem=rs_recv_sems.at[half, j],
                    device_id=(tgt,),
                    device_id_type=pl.DeviceIdType.MESH,
                )
                rdma.start()
                rs_rdmas[half].append(rdma)

        ag_rdmas = []
        reds = []
        for half in (0, 1):
            cols = pl.ds(half * HC, HC)
            for rdma in rs_rdmas[half]:
                rdma.wait_recv()
            red = rs_buf[0, :, cols].astype(jnp.float32)
            for s_ in range(1, N_DEV):
                red = red + rs_buf[s_, :, cols].astype(jnp.float32)
            reds.append(red)
            ag_buf[myrows, cols] = red.astype(bf16)
            for j in range(N_DEV - 1):
                tgt = lax.rem(p + 1 + j, N_DEV)
                rdma = pltpu.make_async_remote_copy(
                    src_ref=ag_buf.at[myrows, cols],
                    dst_ref=ag_buf.at[myrows, cols],
                    send_sem=ag_send_sems.at[half, j],
                    recv_sem=ag_recv_sems.at[half, j],
                    device_id=(tgt,),
                    device_id_type=pl.DeviceIdType.MESH,
                )
                rdma.start()
                ag_rdmas.append(rdma)
        for rdma in ag_rdmas:
            rdma.wait_recv()

        out_vm[0] = ag_buf[...].astype(jnp.float32)
        out_vm[0, myrows, 0:HC] = reds[0]
        out_vm[0, myrows, HC:D] = reds[1]
        store = pltpu.make_async_copy(out_vm, out_ref, store_sem)
        store.start()
        store.wait()

        for rdma in rs_rdmas[0] + rs_rdmas[1]:
            rdma.wait_send()
        for rdma in ag_rdmas:
            rdma.wait_send()

    return pl.pallas_call(
        body,
        out_shape=jax.ShapeDtypeStruct((1, SQ, D), jnp.float32),
        in_specs=[pl.BlockSpec(memory_space=pltpu.VMEM)] * 5,
        out_specs=pl.BlockSpec(memory_space=pltpu.MemorySpace.HBM),
        scratch_shapes=[
            pltpu.VMEM((1, SQ, D), jnp.float32),
            pltpu.VMEM((SQ, D), jnp.bfloat16),
            pltpu.VMEM((N_DEV, CH, D), jnp.bfloat16),
            pltpu.VMEM((SQ, D), jnp.bfloat16),
            pltpu.SemaphoreType.DMA,
            pltpu.SemaphoreType.DMA((2, N_DEV - 1)),
            pltpu.SemaphoreType.DMA((2, N_DEV - 1)),
            pltpu.SemaphoreType.DMA((2, N_DEV - 1)),
            pltpu.SemaphoreType.DMA((2, N_DEV - 1)),
        ],
        compiler_params=pltpu.CompilerParams(collective_id=0),
    )(x, Wq, Wo, Wk, Wv)
